# baseline (speedup 1.0000x reference)
import functools

import jax
import jax.numpy as jnp
from jax import lax
from jax.experimental import pallas as pl
from jax.experimental.pallas import tpu as pltpu
from jax.experimental.pallas import tpu_sc as plsc

B, TGT, SRC, H, V = 4, 256, 2048, 768, 1000

NC, NS = 2, 16
NW = NC * NS
WPB = NW // B
ROWS = TGT // WPB
LANES = 16


@functools.partial(
    pl.kernel,
    out_type=jax.ShapeDtypeStruct((B, TGT, V), jnp.float32),
    mesh=plsc.VectorSubcoreMesh(
        core_axis_name="c", subcore_axis_name="s",
        num_cores=NC, num_subcores=NS,
    ),
    compiler_params=pltpu.CompilerParams(
        needs_layout_passes=False, use_tc_tiling_on_sc=True),
    scratch_types=[
        pltpu.VMEM((SRC,), jnp.int32),
        pltpu.VMEM((ROWS, SRC), jnp.float32),
        pltpu.VMEM((ROWS, V), jnp.float32),
    ],
)
def _sc_logits(ids_hbm, attn_hbm, out_hbm, ids_v, attn_v, acc_v):
    wid = lax.axis_index("s") * NC + lax.axis_index("c")
    b = wid // WPB
    t0 = (wid % WPB) * ROWS

    pltpu.sync_copy(ids_hbm.at[b], ids_v)
    pltpu.sync_copy(attn_hbm.at[pl.ds(b * TGT + t0, ROWS)], attn_v)

    ZUNROLL = 8

    def zero_body(i, _):
        r = i // (V // (LANES * ZUNROLL) + 1)
        c = i % (V // (LANES * ZUNROLL) + 1)
        for u in range(ZUNROLL):
            col = c * (LANES * ZUNROLL) + u * LANES
            acc_v[r, pl.ds(jnp.minimum(col, V - LANES), LANES)] = (
                jnp.zeros((LANES,), jnp.float32))
        return 0

    lax.fori_loop(0, ROWS * (V // (LANES * ZUNROLL) + 1), zero_body, 0)

    def j_body(j, _):
        idv = ids_v[pl.ds(j * LANES, LANES)]
        for r in range(ROWS):
            vals = attn_v[r, pl.ds(j * LANES, LANES)]
            plsc.addupdate_scatter(
                acc_v, [jnp.full((LANES,), r, jnp.int32), idv], vals)
        return 0

    lax.fori_loop(0, SRC // LANES, j_body, 0)

    pltpu.sync_copy(acc_v, out_hbm.at[b, pl.ds(t0, ROWS)])


SRC_TILE = 512
NK = SRC // SRC_TILE


def _pgen_body(attn_ref, src_ref, tgt_ref, w1_ref, w2_ref, bias_ref,
               out_ref, acc_ref):
    k = pl.program_id(1)

    @pl.when(k == 0)
    def _():
        acc_ref[...] = jnp.zeros_like(acc_ref)

    sv = jnp.sum(src_ref[0] * w1_ref[...][:, 0][None, :],
                 axis=1, keepdims=True)
    acc_ref[...] += jnp.sum(attn_ref[0] * sv[:, 0][None, :],
                            axis=1, keepdims=True)

    @pl.when(k == NK - 1)
    def _():
        t2 = jnp.sum(tgt_ref[0] * w2_ref[...][:, 0][None, :],
                     axis=1, keepdims=True)
        z = acc_ref[...] + t2 + bias_ref[0, 0]
        out_ref[0, 0] = jax.nn.sigmoid(z)[:, 0]


_pgen_call = pl.pallas_call(
    _pgen_body,
    grid=(B, NK),
    in_specs=[
        pl.BlockSpec((1, TGT, SRC_TILE), lambda b, k: (b, 0, k)),
        pl.BlockSpec((1, SRC_TILE, H), lambda b, k: (b, k, 0)),
        pl.BlockSpec((1, TGT, H), lambda b, k: (b, 0, 0)),
        pl.BlockSpec((H, 1), lambda b, k: (0, 0)),
        pl.BlockSpec((H, 1), lambda b, k: (0, 0)),
        pl.BlockSpec((1, 1), lambda b, k: (0, 0)),
    ],
    out_specs=pl.BlockSpec((1, 1, TGT), lambda b, k: (b, 0, 0)),
    out_shape=jax.ShapeDtypeStruct((B, 1, TGT), jnp.float32),
    scratch_shapes=[pltpu.VMEM((TGT, 1), jnp.float32)],
)


def kernel(input_ids_to_copy, cross_attentions, src_hidden_states,
           tgt_hidden_states, w_pgen, b_pgen):
    ids2d = input_ids_to_copy.astype(jnp.int32)
    attn2d = cross_attentions.reshape(B * TGT, SRC)

    logits = _sc_logits(ids2d, attn2d)

    w1 = w_pgen[:H]
    w2 = w_pgen[H:]
    p_gen = _pgen_call(
        cross_attentions, src_hidden_states, tgt_hidden_states,
        w1, w2, b_pgen.reshape(1, 1),
    ).reshape(B, TGT, 1)

    return (p_gen, logits)


# speedup vs baseline: 1.0235x; 1.0235x over previous
import functools

import jax
import jax.numpy as jnp
from jax import lax
from jax.experimental import pallas as pl
from jax.experimental.pallas import tpu as pltpu
from jax.experimental.pallas import tpu_sc as plsc

B, TGT, SRC, H, V = 4, 256, 2048, 768, 1000

NC, NS = 2, 16
NW = NC * NS
WPB = NW // B
ROWS = TGT // WPB
LANES = 16


@functools.partial(
    pl.kernel,
    out_type=jax.ShapeDtypeStruct((B, TGT, V), jnp.float32),
    mesh=plsc.VectorSubcoreMesh(
        core_axis_name="c", subcore_axis_name="s",
        num_cores=NC, num_subcores=NS,
    ),
    compiler_params=pltpu.CompilerParams(
        needs_layout_passes=False, use_tc_tiling_on_sc=True),
    scratch_types=[
        pltpu.VMEM((SRC,), jnp.int32),
        pltpu.VMEM((ROWS, SRC), jnp.float32),
        pltpu.VMEM((ROWS, V), jnp.float32),
        pltpu.SemaphoreType.DMA,
        pltpu.SemaphoreType.DMA,
        pltpu.SemaphoreType.DMA,
        pltpu.SemaphoreType.DMA,
        pltpu.SemaphoreType.DMA,
    ],
)
def _sc_logits(ids_hbm, attn_hbm, out_hbm, ids_v, attn_v, acc_v,
               sem_a0, sem_a1, sem_a2, sem_a3, sem_o):
    wid = lax.axis_index("s") * NC + lax.axis_index("c")
    b = wid // WPB
    t0 = (wid % WPB) * ROWS

    pltpu.sync_copy(ids_hbm.at[b], ids_v)

    CH = 4
    RPC = ROWS // CH
    sems = [sem_a0, sem_a1, sem_a2, sem_a3]
    loads = [
        pltpu.async_copy(
            attn_hbm.at[pl.ds(b * TGT + t0 + c * RPC, RPC)],
            attn_v.at[pl.ds(c * RPC, RPC)], sems[c])
        for c in range(CH)
    ]

    ZUNROLL = 8
    NZC = V // (LANES * ZUNROLL) + 1

    def zero_body(i, _):
        r = i // NZC
        c = i % NZC
        for u in range(ZUNROLL):
            col = c * (LANES * ZUNROLL) + u * LANES
            acc_v[r, pl.ds(jnp.minimum(col, V - LANES), LANES)] = (
                jnp.zeros((LANES,), jnp.float32))
        return 0

    lax.fori_loop(0, ROWS * NZC, zero_body, 0)

    stores = []
    for c in range(CH):
        loads[c].wait()

        def j_body(j, _, c=c):
            idv = ids_v[pl.ds(j * LANES, LANES)]
            for r8 in range(RPC):
                r = c * RPC + r8
                vals = attn_v[r, pl.ds(j * LANES, LANES)]
                plsc.addupdate_scatter(
                    acc_v, [jnp.full((LANES,), r, jnp.int32), idv], vals)
            return 0

        lax.fori_loop(0, SRC // LANES, j_body, 0)
        stores.append(pltpu.async_copy(
            acc_v.at[pl.ds(c * RPC, RPC)],
            out_hbm.at[b, pl.ds(t0 + c * RPC, RPC)], sem_o))

    for st in stores:
        st.wait()


SRC_TILE = 512
NK = SRC // SRC_TILE


def _pgen_body(attn_ref, src_ref, tgt_ref, w1_ref, w2_ref, bias_ref,
               out_ref, ctx_ref):
    k = pl.program_id(1)

    @pl.when(k == 0)
    def _():
        ctx_ref[...] = jnp.zeros_like(ctx_ref)

    ctx_ref[...] += jnp.dot(attn_ref[0], src_ref[0],
                            preferred_element_type=jnp.float32)

    @pl.when(k == NK - 1)
    def _():
        z = (jnp.dot(ctx_ref[...], w1_ref[...],
                     preferred_element_type=jnp.float32)
             + jnp.dot(tgt_ref[0], w2_ref[...],
                       preferred_element_type=jnp.float32)
             + bias_ref[0, 0])
        out_ref[0, 0] = jax.nn.sigmoid(z)[:, 0]


_pgen_call = pl.pallas_call(
    _pgen_body,
    grid=(B, NK),
    in_specs=[
        pl.BlockSpec((1, TGT, SRC_TILE), lambda b, k: (b, 0, k)),
        pl.BlockSpec((1, SRC_TILE, H), lambda b, k: (b, k, 0)),
        pl.BlockSpec((1, TGT, H), lambda b, k: (b, 0, 0)),
        pl.BlockSpec((H, 1), lambda b, k: (0, 0)),
        pl.BlockSpec((H, 1), lambda b, k: (0, 0)),
        pl.BlockSpec((1, 1), lambda b, k: (0, 0)),
    ],
    out_specs=pl.BlockSpec((1, 1, TGT), lambda b, k: (b, 0, 0)),
    out_shape=jax.ShapeDtypeStruct((B, 1, TGT), jnp.float32),
    scratch_shapes=[pltpu.VMEM((TGT, H), jnp.float32)],
)


def kernel(input_ids_to_copy, cross_attentions, src_hidden_states,
           tgt_hidden_states, w_pgen, b_pgen):
    ids2d = input_ids_to_copy.astype(jnp.int32)
    attn2d = cross_attentions.reshape(B * TGT, SRC)

    logits = _sc_logits(ids2d, attn2d)

    w1 = w_pgen[:H]
    w2 = w_pgen[H:]
    p_gen = _pgen_call(
        cross_attentions, src_hidden_states, tgt_hidden_states,
        w1, w2, b_pgen.reshape(1, 1),
    ).reshape(B, TGT, 1)

    return (p_gen, logits)
